# Initial kernel scaffold; baseline (speedup 1.0000x reference)
import jax
import jax.numpy as jnp
from jax.experimental import pallas as pl


_BL = 512


def _body(x_ref, tab_ref, o_ref):
    j = pl.program_id(1)
    pe = tab_ref[pl.ds(j * _BL + 1, _BL), :]
    o_ref[...] = x_ref[...] + pe[None]


def kernel(x, pos_table):
    B, L, D = x.shape
    T = pos_table.shape[0]
    grid = (B, L // _BL)
    return pl.pallas_call(
        _body,
        grid=grid,
        in_specs=[
            pl.BlockSpec((1, _BL, D), lambda b, j: (b, j, 0)),
            pl.BlockSpec((T, D), lambda b, j: (0, 0)),
        ],
        out_specs=pl.BlockSpec((1, _BL, D), lambda b, j: (b, j, 0)),
        out_shape=jax.ShapeDtypeStruct(x.shape, x.dtype),
    )(x, pos_table)


# speedup vs baseline: 2.8828x; 2.8828x over previous
import jax
import jax.numpy as jnp
from jax.experimental import pallas as pl


_BL = 512


def _body(x_ref, tab_ref, o_ref):
    j = pl.program_id(1)
    win = tab_ref[pl.ds(j * _BL, _BL + 8), :]
    pe = win[1:_BL + 1]
    o_ref[...] = x_ref[...] + pe[None]


def kernel(x, pos_table):
    B, L, D = x.shape
    Tp = L + 8
    tab = jnp.pad(pos_table, ((0, Tp - pos_table.shape[0]), (0, 0)))
    grid = (B, L // _BL)
    return pl.pallas_call(
        _body,
        grid=grid,
        in_specs=[
            pl.BlockSpec((1, _BL, D), lambda b, j: (b, j, 0)),
            pl.BlockSpec((Tp, D), lambda b, j: (0, 0)),
        ],
        out_specs=pl.BlockSpec((1, _BL, D), lambda b, j: (b, j, 0)),
        out_shape=jax.ShapeDtypeStruct(x.shape, x.dtype),
    )(x, pos_table)
